# Initial kernel scaffold; baseline (speedup 1.0000x reference)
import jax
import jax.numpy as jnp
from jax.experimental import pallas as pl


def kernel(hidden, W1, b1, W2, b2):
    raise NotImplementedError("write your pallas kernel here")


# speedup vs baseline: 2.0736x; 2.0736x over previous
import functools

import numpy as np
import jax
import jax.numpy as jnp
from jax.experimental import pallas as pl

_KS = 3
_NEG = -10.0


def _pos_emb(L, D):
    pos = np.arange(L)[:, None].astype(np.float64)
    i = np.arange(D)[None, :].astype(np.float64)
    angle = pos / np.power(10000.0, (2.0 * (i // 2)) / D)
    pe = np.zeros((L, D), dtype=np.float64)
    pe[:, 0::2] = np.sin(angle[:, 0::2])
    pe[:, 1::2] = np.cos(angle[:, 1::2])
    return jnp.asarray(pe, dtype=jnp.float32)


def _boundary_kernel(hid_ref, w1_ref, b1_ref, w2_ref, b2_ref,
                     seg_ref, nb_ref, adj_ref):
    x = hid_ref[0]
    L = x.shape[0]
    acc = None
    for k in range(_KS):
        a = jax.lax.dot_general(w1_ref[k], x, (((1,), (1,)), ((), ())),
                                preferred_element_type=jnp.float32)
        if k:
            a = jnp.roll(a, -k, axis=1)
        acc = a if acc is None else acc + a
    r = jnp.maximum(acc + b1_ref[...], 0.0)
    strided = jnp.dot(w2_ref[...], r,
                      preferred_element_type=jnp.float32) + b2_ref[0, 0]
    full = jnp.roll(strided, _KS - 1, axis=1)
    lane = jax.lax.broadcasted_iota(jnp.int32, (1, L), 1)
    full = jnp.where(lane < _KS - 1, _NEG, full)
    h = (full > 0.0).astype(jnp.float32)
    nb_ref[0] = jnp.sum(h, keepdims=True)
    hprev = jnp.where(lane < 1, 0.0, jnp.roll(h, 1, axis=1))
    adj_ref[0] = jnp.sum(h * hprev, keepdims=True)
    seg = jnp.zeros((1, L), jnp.float32)
    C = 128
    for j in range(L // C):
        hj = h[:, j * C:(j + 1) * C]
        row = jax.lax.broadcasted_iota(jnp.int32, (C, L), 0) + (j * C)
        col = jax.lax.broadcasted_iota(jnp.int32, (C, L), 1)
        m = (row < col).astype(jnp.float32)
        seg = seg + jnp.dot(hj, m, preferred_element_type=jnp.float32)
    seg_ref[0] = seg


def _pool_kernel(hid_ref, seg_ref, pe_ref, pooled_ref, cnt_ref, *, ts):
    s = pl.program_id(1)
    x = hid_ref[0]
    srow = seg_ref[0]
    sidx = (jax.lax.broadcasted_iota(jnp.int32, (ts, 1), 0)
            + s * ts).astype(jnp.float32)
    pm = (sidx == srow).astype(jnp.float32)
    sums = jnp.dot(pm, x, preferred_element_type=jnp.float32)
    cnt = jnp.sum(pm, axis=1, keepdims=True)
    pooled_ref[0] = sums / jnp.maximum(cnt, 1.0) + pe_ref[...]
    cnt_ref[0] = cnt


def _stats_kernel(cnt_ref, nb_ref, adj_ref, out_ref):
    c = cnt_ref[...]
    valid = (c > 0.0).astype(jnp.float32)
    n_valid = jnp.maximum(jnp.sum(valid), 1.0)
    mean = jnp.sum(c * valid) / n_valid
    var = jnp.sum(valid * (c - mean) ** 2) / n_valid
    cv = jnp.sqrt(var) / jnp.maximum(mean, 1e-6)
    nb = jnp.sum(nb_ref[...])
    adj = jnp.sum(adj_ref[...])
    adj_pct = 100.0 * adj / jnp.maximum(nb, 1.0)
    lane = jax.lax.broadcasted_iota(jnp.int32, (1, 8), 1)
    out_ref[...] = jnp.where(
        lane == 0, nb, jnp.where(lane == 1, cv,
                                 jnp.where(lane == 2, adj_pct, 0.0)))


def kernel(hidden, W1, b1, W2, b2):
    B, L, D = hidden.shape
    H = W1.shape[0]
    HP = 128
    W1p = jnp.zeros((_KS, HP, D), jnp.float32).at[:, :H, :].set(
        jnp.transpose(W1, (2, 0, 1)))
    b1p = jnp.zeros((HP, 1), jnp.float32).at[:H, 0].set(b1)
    w2p = jnp.zeros((1, HP), jnp.float32).at[0, :H].set(W2[0, :, 0])
    b2r = b2.reshape(1, 1).astype(jnp.float32)

    seg, nb, adj = pl.pallas_call(
        _boundary_kernel,
        grid=(B,),
        in_specs=[
            pl.BlockSpec((1, L, D), lambda b: (b, 0, 0)),
            pl.BlockSpec((_KS, HP, D), lambda b: (0, 0, 0)),
            pl.BlockSpec((HP, 1), lambda b: (0, 0)),
            pl.BlockSpec((1, HP), lambda b: (0, 0)),
            pl.BlockSpec((1, 1), lambda b: (0, 0)),
        ],
        out_specs=[
            pl.BlockSpec((1, 1, L), lambda b: (b, 0, 0)),
            pl.BlockSpec((1, 1, 1), lambda b: (b, 0, 0)),
            pl.BlockSpec((1, 1, 1), lambda b: (b, 0, 0)),
        ],
        out_shape=[
            jax.ShapeDtypeStruct((B, 1, L), jnp.float32),
            jax.ShapeDtypeStruct((B, 1, 1), jnp.float32),
            jax.ShapeDtypeStruct((B, 1, 1), jnp.float32),
        ],
    )(hidden, W1p, b1p, w2p, b2r)

    TS = 512
    pe = _pos_emb(L, D)
    pooled, cnt3 = pl.pallas_call(
        functools.partial(_pool_kernel, ts=TS),
        grid=(B, L // TS),
        in_specs=[
            pl.BlockSpec((1, L, D), lambda b, s: (b, 0, 0)),
            pl.BlockSpec((1, 1, L), lambda b, s: (b, 0, 0)),
            pl.BlockSpec((TS, D), lambda b, s: (s, 0)),
        ],
        out_specs=[
            pl.BlockSpec((1, TS, D), lambda b, s: (b, s, 0)),
            pl.BlockSpec((1, TS, 1), lambda b, s: (b, s, 0)),
        ],
        out_shape=[
            jax.ShapeDtypeStruct((B, L, D), jnp.float32),
            jax.ShapeDtypeStruct((B, L, 1), jnp.float32),
        ],
    )(hidden, seg, pe)

    counts = cnt3.reshape(B, L)
    stats = pl.pallas_call(
        _stats_kernel,
        out_shape=jax.ShapeDtypeStruct((1, 8), jnp.float32),
    )(counts, nb, adj)

    loss = jnp.asarray(0.0, dtype=jnp.float32)
    total_positions = jnp.asarray(float(B * L), dtype=jnp.float32)
    return (pooled, loss, stats[0, 0], total_positions, stats[0, 1],
            stats[0, 2])


# speedup vs baseline: 2.5674x; 1.2381x over previous
import functools

import numpy as np
import jax
import jax.numpy as jnp
from jax.experimental import pallas as pl

_KS = 3
_NEG = -10.0


def _pos_emb(L, D):
    pos = np.arange(L)[:, None].astype(np.float64)
    i = np.arange(D)[None, :].astype(np.float64)
    angle = pos / np.power(10000.0, (2.0 * (i // 2)) / D)
    pe = np.zeros((L, D), dtype=np.float64)
    pe[:, 0::2] = np.sin(angle[:, 0::2])
    pe[:, 1::2] = np.cos(angle[:, 1::2])
    return jnp.asarray(pe, dtype=jnp.float32)


def _tri_mask(L):
    m = np.arange(L)[:, None] < np.arange(L)[None, :]
    return jnp.asarray(m, dtype=jnp.bfloat16)


def _fused_kernel(hid_ref, w1_ref, b1_ref, w2_ref, b2_ref, m_ref, pe_ref,
                  pooled_ref, cnt_ref, nb_ref, adj_ref, *, ts):
    x = hid_ref[0]
    L = x.shape[0]
    acc = None
    for k in range(_KS):
        a = jax.lax.dot_general(w1_ref[k], x, (((1,), (1,)), ((), ())),
                                preferred_element_type=jnp.float32)
        if k:
            a = jnp.roll(a, -k, axis=1)
        acc = a if acc is None else acc + a
    r = jnp.maximum(acc + b1_ref[...], 0.0)
    strided = jnp.dot(w2_ref[...], r,
                      preferred_element_type=jnp.float32) + b2_ref[0, 0]
    full = jnp.roll(strided, _KS - 1, axis=1)
    lane = jax.lax.broadcasted_iota(jnp.int32, (1, L), 1)
    full = jnp.where(lane < _KS - 1, _NEG, full)
    h = (full > 0.0).astype(jnp.float32)
    nb_ref[0] = jnp.sum(h, keepdims=True)
    hprev = jnp.where(lane < 1, 0.0, jnp.roll(h, 1, axis=1))
    adj_ref[0] = jnp.sum(h * hprev, keepdims=True)
    seg = jnp.dot(h.astype(jnp.bfloat16), m_ref[...],
                  preferred_element_type=jnp.float32)
    for st in range(L // ts):
        sidx = (jax.lax.broadcasted_iota(jnp.int32, (ts, 1), 0)
                + st * ts).astype(jnp.float32)
        pm = (sidx == seg).astype(jnp.float32)
        sums = jnp.dot(pm, x, preferred_element_type=jnp.float32)
        cnt = jnp.sum(pm, axis=1, keepdims=True)
        sl = slice(st * ts, (st + 1) * ts)
        pooled_ref[0, sl, :] = sums / jnp.maximum(cnt, 1.0) + pe_ref[sl, :]
        cnt_ref[0, sl] = cnt


def _stats_kernel(cnt_ref, nb_ref, adj_ref, out_ref):
    c = cnt_ref[...]
    valid = (c > 0.0).astype(jnp.float32)
    n_valid = jnp.maximum(jnp.sum(valid), 1.0)
    mean = jnp.sum(c * valid) / n_valid
    var = jnp.sum(valid * (c - mean) ** 2) / n_valid
    cv = jnp.sqrt(var) / jnp.maximum(mean, 1e-6)
    nb = jnp.sum(nb_ref[...])
    adj = jnp.sum(adj_ref[...])
    adj_pct = 100.0 * adj / jnp.maximum(nb, 1.0)
    lane = jax.lax.broadcasted_iota(jnp.int32, (1, 8), 1)
    out_ref[...] = jnp.where(
        lane == 0, nb, jnp.where(lane == 1, cv,
                                 jnp.where(lane == 2, adj_pct, 0.0)))


def kernel(hidden, W1, b1, W2, b2):
    B, L, D = hidden.shape
    H = W1.shape[0]
    HP = 128
    W1p = jnp.zeros((_KS, HP, D), jnp.float32).at[:, :H, :].set(
        jnp.transpose(W1, (2, 0, 1)))
    b1p = jnp.zeros((HP, 1), jnp.float32).at[:H, 0].set(b1)
    w2p = jnp.zeros((1, HP), jnp.float32).at[0, :H].set(W2[0, :, 0])
    b2r = b2.reshape(1, 1).astype(jnp.float32)
    tri = _tri_mask(L)
    pe = _pos_emb(L, D)

    TS = 512
    pooled, cnt3, nb, adj = pl.pallas_call(
        functools.partial(_fused_kernel, ts=TS),
        grid=(B,),
        in_specs=[
            pl.BlockSpec((1, L, D), lambda b: (b, 0, 0)),
            pl.BlockSpec((_KS, HP, D), lambda b: (0, 0, 0)),
            pl.BlockSpec((HP, 1), lambda b: (0, 0)),
            pl.BlockSpec((1, HP), lambda b: (0, 0)),
            pl.BlockSpec((1, 1), lambda b: (0, 0)),
            pl.BlockSpec((L, L), lambda b: (0, 0)),
            pl.BlockSpec((L, D), lambda b: (0, 0)),
        ],
        out_specs=[
            pl.BlockSpec((1, L, D), lambda b: (b, 0, 0)),
            pl.BlockSpec((1, L, 1), lambda b: (b, 0, 0)),
            pl.BlockSpec((1, 1, 1), lambda b: (b, 0, 0)),
            pl.BlockSpec((1, 1, 1), lambda b: (b, 0, 0)),
        ],
        out_shape=[
            jax.ShapeDtypeStruct((B, L, D), jnp.float32),
            jax.ShapeDtypeStruct((B, L, 1), jnp.float32),
            jax.ShapeDtypeStruct((B, 1, 1), jnp.float32),
            jax.ShapeDtypeStruct((B, 1, 1), jnp.float32),
        ],
    )(hidden, W1p, b1p, w2p, b2r, tri, pe)

    counts = cnt3.reshape(B, L)
    stats = pl.pallas_call(
        _stats_kernel,
        out_shape=jax.ShapeDtypeStruct((1, 8), jnp.float32),
    )(counts, nb, adj)

    loss = jnp.asarray(0.0, dtype=jnp.float32)
    total_positions = jnp.asarray(float(B * L), dtype=jnp.float32)
    return (pooled, loss, stats[0, 0], total_positions, stats[0, 1],
            stats[0, 2])


# speedup vs baseline: 2.9583x; 1.1523x over previous
import functools

import numpy as np
import jax
import jax.numpy as jnp
from jax.experimental import pallas as pl

_KS = 3
_NEG = -10.0


def _pos_emb(L, D):
    pos = np.arange(L)[:, None].astype(np.float64)
    i = np.arange(D)[None, :].astype(np.float64)
    angle = pos / np.power(10000.0, (2.0 * (i // 2)) / D)
    pe = np.zeros((L, D), dtype=np.float64)
    pe[:, 0::2] = np.sin(angle[:, 0::2])
    pe[:, 1::2] = np.cos(angle[:, 1::2])
    return jnp.asarray(pe, dtype=jnp.float32)


def _tri_mask(L):
    m = np.arange(L)[:, None] < np.arange(L)[None, :]
    return jnp.asarray(m, dtype=jnp.bfloat16)


def _fused_kernel(hid_ref, w1_ref, b1_ref, w2_ref, b2_ref, m_ref, pe_ref,
                  pooled_ref, cnt_ref, nb_ref, adj_ref, *, ts):
    x = hid_ref[0]
    L = x.shape[0]
    acc = None
    for k in range(_KS):
        a = jax.lax.dot_general(w1_ref[k], x, (((1,), (1,)), ((), ())),
                                preferred_element_type=jnp.float32)
        if k:
            a = jnp.roll(a, -k, axis=1)
        acc = a if acc is None else acc + a
    r = jnp.maximum(acc + b1_ref[...], 0.0)
    strided = jnp.dot(w2_ref[...], r,
                      preferred_element_type=jnp.float32) + b2_ref[0, 0]
    full = jnp.roll(strided, _KS - 1, axis=1)
    lane = jax.lax.broadcasted_iota(jnp.int32, (1, L), 1)
    full = jnp.where(lane < _KS - 1, _NEG, full)
    h = (full > 0.0).astype(jnp.float32)
    nb_ref[0] = jnp.sum(h, keepdims=True)
    hprev = jnp.where(lane < 1, 0.0, jnp.roll(h, 1, axis=1))
    adj_ref[0] = jnp.sum(h * hprev, keepdims=True)
    seg = jnp.dot(h.astype(jnp.bfloat16), m_ref[...],
                  preferred_element_type=jnp.float32)
    x_hi = x.astype(jnp.bfloat16)
    x_lo = (x - x_hi.astype(jnp.float32)).astype(jnp.bfloat16)
    nbs = jnp.sum(h)
    for st in range(L // ts):
        sl = slice(st * ts, (st + 1) * ts)

        def _run(sl=sl, st=st):
            sidx = (jax.lax.broadcasted_iota(jnp.int32, (ts, 1), 0)
                    + st * ts).astype(jnp.float32)
            pmf = (sidx == seg).astype(jnp.float32)
            pm = pmf.astype(jnp.bfloat16)
            sums = (jnp.dot(pm, x_hi, preferred_element_type=jnp.float32)
                    + jnp.dot(pm, x_lo, preferred_element_type=jnp.float32))
            cnt = jnp.sum(pmf, axis=1, keepdims=True)
            pooled_ref[0, sl, :] = sums / jnp.maximum(cnt, 1.0) + pe_ref[sl, :]
            cnt_ref[0, sl] = cnt

        if st == 0:
            _run()
        else:
            pl.when(nbs >= float(st * ts))(_run)

            @pl.when(nbs < float(st * ts))
            def _skip(sl=sl):
                pooled_ref[0, sl, :] = pe_ref[sl, :]
                cnt_ref[0, sl] = jnp.zeros((ts, 1), jnp.float32)


def _stats_kernel(cnt_ref, nb_ref, adj_ref, out_ref):
    c = cnt_ref[...]
    valid = (c > 0.0).astype(jnp.float32)
    n_valid = jnp.maximum(jnp.sum(valid), 1.0)
    mean = jnp.sum(c * valid) / n_valid
    var = jnp.sum(valid * (c - mean) ** 2) / n_valid
    cv = jnp.sqrt(var) / jnp.maximum(mean, 1e-6)
    nb = jnp.sum(nb_ref[...])
    adj = jnp.sum(adj_ref[...])
    adj_pct = 100.0 * adj / jnp.maximum(nb, 1.0)
    lane = jax.lax.broadcasted_iota(jnp.int32, (1, 8), 1)
    out_ref[...] = jnp.where(
        lane == 0, nb, jnp.where(lane == 1, cv,
                                 jnp.where(lane == 2, adj_pct, 0.0)))


def kernel(hidden, W1, b1, W2, b2):
    B, L, D = hidden.shape
    H = W1.shape[0]
    HP = 128
    W1p = jnp.zeros((_KS, HP, D), jnp.float32).at[:, :H, :].set(
        jnp.transpose(W1, (2, 0, 1)))
    b1p = jnp.zeros((HP, 1), jnp.float32).at[:H, 0].set(b1)
    w2p = jnp.zeros((1, HP), jnp.float32).at[0, :H].set(W2[0, :, 0])
    b2r = b2.reshape(1, 1).astype(jnp.float32)
    tri = _tri_mask(L)
    pe = _pos_emb(L, D)

    TS = 512
    pooled, cnt3, nb, adj = pl.pallas_call(
        functools.partial(_fused_kernel, ts=TS),
        grid=(B,),
        in_specs=[
            pl.BlockSpec((1, L, D), lambda b: (b, 0, 0)),
            pl.BlockSpec((_KS, HP, D), lambda b: (0, 0, 0)),
            pl.BlockSpec((HP, 1), lambda b: (0, 0)),
            pl.BlockSpec((1, HP), lambda b: (0, 0)),
            pl.BlockSpec((1, 1), lambda b: (0, 0)),
            pl.BlockSpec((L, L), lambda b: (0, 0)),
            pl.BlockSpec((L, D), lambda b: (0, 0)),
        ],
        out_specs=[
            pl.BlockSpec((1, L, D), lambda b: (b, 0, 0)),
            pl.BlockSpec((1, L, 1), lambda b: (b, 0, 0)),
            pl.BlockSpec((1, 1, 1), lambda b: (b, 0, 0)),
            pl.BlockSpec((1, 1, 1), lambda b: (b, 0, 0)),
        ],
        out_shape=[
            jax.ShapeDtypeStruct((B, L, D), jnp.float32),
            jax.ShapeDtypeStruct((B, L, 1), jnp.float32),
            jax.ShapeDtypeStruct((B, 1, 1), jnp.float32),
            jax.ShapeDtypeStruct((B, 1, 1), jnp.float32),
        ],
    )(hidden, W1p, b1p, w2p, b2r, tri, pe)

    counts = cnt3.reshape(B, L)
    stats = pl.pallas_call(
        _stats_kernel,
        out_shape=jax.ShapeDtypeStruct((1, 8), jnp.float32),
    )(counts, nb, adj)

    loss = jnp.asarray(0.0, dtype=jnp.float32)
    total_positions = jnp.asarray(float(B * L), dtype=jnp.float32)
    return (pooled, loss, stats[0, 0], total_positions, stats[0, 1],
            stats[0, 2])


# speedup vs baseline: 3.1877x; 1.0775x over previous
import functools

import numpy as np
import jax
import jax.numpy as jnp
from jax.experimental import pallas as pl

_KS = 3
_NEG = -10.0


def _pos_emb(L, D):
    pos = np.arange(L)[:, None].astype(np.float64)
    i = np.arange(D)[None, :].astype(np.float64)
    angle = pos / np.power(10000.0, (2.0 * (i // 2)) / D)
    pe = np.zeros((L, D), dtype=np.float64)
    pe[:, 0::2] = np.sin(angle[:, 0::2])
    pe[:, 1::2] = np.cos(angle[:, 1::2])
    return jnp.asarray(pe, dtype=jnp.float32)


def _fused_kernel(hid_ref, w1_ref, b1_ref, w2_ref, b2_ref, pe_ref,
                  pooled_ref, cnt_ref, nb_ref, adj_ref, *, ts):
    x = hid_ref[0]
    L = x.shape[0]
    acc = None
    for k in range(_KS):
        a = jax.lax.dot_general(w1_ref[k], x, (((1,), (1,)), ((), ())),
                                preferred_element_type=jnp.float32)
        if k:
            a = jnp.roll(a, -k, axis=1)
        acc = a if acc is None else acc + a
    r = jnp.maximum(acc + b1_ref[...], 0.0)
    strided = jnp.dot(w2_ref[...], r,
                      preferred_element_type=jnp.float32) + b2_ref[0, 0]
    full = jnp.roll(strided, _KS - 1, axis=1)
    lane = jax.lax.broadcasted_iota(jnp.int32, (1, L), 1)
    full = jnp.where(lane < _KS - 1, _NEG, full)
    h = (full > 0.0).astype(jnp.float32)
    nb_ref[0] = jnp.sum(h, keepdims=True)
    hprev = jnp.where(lane < 1, 0.0, jnp.roll(h, 1, axis=1))
    adj_ref[0] = jnp.sum(h * hprev, keepdims=True)
    C = 128
    nchunk = L // C
    hb = h.astype(jnp.bfloat16)
    t128 = (jax.lax.broadcasted_iota(jnp.int32, (C, C), 0)
            < jax.lax.broadcasted_iota(jnp.int32, (C, C), 1)).astype(
                jnp.bfloat16)
    fine = jnp.concatenate(
        [jnp.dot(hb[:, j * C:(j + 1) * C], t128,
                 preferred_element_type=jnp.float32)
         for j in range(nchunk)], axis=1)
    bmask = (jax.lax.broadcasted_iota(jnp.int32, (L, nchunk), 0) // C
             == jax.lax.broadcasted_iota(jnp.int32, (L, nchunk), 1)).astype(
                 jnp.bfloat16)
    tot = jnp.dot(hb, bmask, preferred_element_type=jnp.float32)
    qmask = (jax.lax.broadcasted_iota(jnp.int32, (nchunk, L), 0)
             < jax.lax.broadcasted_iota(jnp.int32, (nchunk, L), 1) // C
             ).astype(jnp.bfloat16)
    offs = jnp.dot(tot.astype(jnp.bfloat16), qmask,
                   preferred_element_type=jnp.float32)
    seg = fine + offs
    x_hi = x.astype(jnp.bfloat16)
    x_lo = (x - x_hi.astype(jnp.float32)).astype(jnp.bfloat16)
    nbs = jnp.sum(h)
    for st in range(L // ts):
        sl = slice(st * ts, (st + 1) * ts)

        def _run(sl=sl, st=st):
            sidx = (jax.lax.broadcasted_iota(jnp.int32, (ts, 1), 0)
                    + st * ts).astype(jnp.float32)
            pmf = (sidx == seg).astype(jnp.float32)
            pm = pmf.astype(jnp.bfloat16)
            sums = (jnp.dot(pm, x_hi, preferred_element_type=jnp.float32)
                    + jnp.dot(pm, x_lo, preferred_element_type=jnp.float32))
            cnt = jnp.sum(pmf, axis=1, keepdims=True)
            pooled_ref[0, sl, :] = sums / jnp.maximum(cnt, 1.0) + pe_ref[sl, :]
            cnt_ref[0, sl] = cnt

        if st == 0:
            _run()
        else:
            pl.when(nbs >= float(st * ts))(_run)

            @pl.when(nbs < float(st * ts))
            def _skip(sl=sl):
                pooled_ref[0, sl, :] = pe_ref[sl, :]
                cnt_ref[0, sl] = jnp.zeros((ts, 1), jnp.float32)


def _stats_kernel(cnt_ref, nb_ref, adj_ref, out_ref):
    c = cnt_ref[...]
    valid = (c > 0.0).astype(jnp.float32)
    n_valid = jnp.maximum(jnp.sum(valid), 1.0)
    mean = jnp.sum(c * valid) / n_valid
    var = jnp.sum(valid * (c - mean) ** 2) / n_valid
    cv = jnp.sqrt(var) / jnp.maximum(mean, 1e-6)
    nb = jnp.sum(nb_ref[...])
    adj = jnp.sum(adj_ref[...])
    adj_pct = 100.0 * adj / jnp.maximum(nb, 1.0)
    lane = jax.lax.broadcasted_iota(jnp.int32, (1, 8), 1)
    out_ref[...] = jnp.where(
        lane == 0, nb, jnp.where(lane == 1, cv,
                                 jnp.where(lane == 2, adj_pct, 0.0)))


def kernel(hidden, W1, b1, W2, b2):
    B, L, D = hidden.shape
    H = W1.shape[0]
    HP = 128
    W1p = jnp.zeros((_KS, HP, D), jnp.float32).at[:, :H, :].set(
        jnp.transpose(W1, (2, 0, 1)))
    b1p = jnp.zeros((HP, 1), jnp.float32).at[:H, 0].set(b1)
    w2p = jnp.zeros((1, HP), jnp.float32).at[0, :H].set(W2[0, :, 0])
    b2r = b2.reshape(1, 1).astype(jnp.float32)
    pe = _pos_emb(L, D)

    TS = 512
    pooled, cnt3, nb, adj = pl.pallas_call(
        functools.partial(_fused_kernel, ts=TS),
        grid=(B,),
        in_specs=[
            pl.BlockSpec((1, L, D), lambda b: (b, 0, 0)),
            pl.BlockSpec((_KS, HP, D), lambda b: (0, 0, 0)),
            pl.BlockSpec((HP, 1), lambda b: (0, 0)),
            pl.BlockSpec((1, HP), lambda b: (0, 0)),
            pl.BlockSpec((1, 1), lambda b: (0, 0)),
            pl.BlockSpec((L, D), lambda b: (0, 0)),
        ],
        out_specs=[
            pl.BlockSpec((1, L, D), lambda b: (b, 0, 0)),
            pl.BlockSpec((1, L, 1), lambda b: (b, 0, 0)),
            pl.BlockSpec((1, 1, 1), lambda b: (b, 0, 0)),
            pl.BlockSpec((1, 1, 1), lambda b: (b, 0, 0)),
        ],
        out_shape=[
            jax.ShapeDtypeStruct((B, L, D), jnp.float32),
            jax.ShapeDtypeStruct((B, L, 1), jnp.float32),
            jax.ShapeDtypeStruct((B, 1, 1), jnp.float32),
            jax.ShapeDtypeStruct((B, 1, 1), jnp.float32),
        ],
    )(hidden, W1p, b1p, w2p, b2r, pe)

    counts = cnt3.reshape(B, L)
    stats = pl.pallas_call(
        _stats_kernel,
        out_shape=jax.ShapeDtypeStruct((1, 8), jnp.float32),
    )(counts, nb, adj)

    loss = jnp.asarray(0.0, dtype=jnp.float32)
    total_positions = jnp.asarray(float(B * L), dtype=jnp.float32)
    return (pooled, loss, stats[0, 0], total_positions, stats[0, 1],
            stats[0, 2])


# speedup vs baseline: 3.4653x; 1.0871x over previous
import functools

import numpy as np
import jax
import jax.numpy as jnp
from jax.experimental import pallas as pl

_KS = 3
_NEG = -10.0


def _pos_emb(L, D):
    pos = np.arange(L)[:, None].astype(np.float64)
    i = np.arange(D)[None, :].astype(np.float64)
    angle = pos / np.power(10000.0, (2.0 * (i // 2)) / D)
    pe = np.zeros((L, D), dtype=np.float64)
    pe[:, 0::2] = np.sin(angle[:, 0::2])
    pe[:, 1::2] = np.cos(angle[:, 1::2])
    return jnp.asarray(pe, dtype=jnp.float32)


def _fused_kernel(hid_ref, w1_ref, b1_ref, w2_ref, b2_ref, pe_ref,
                  pooled_ref, cnt_ref, nb_ref, adj_ref, *, ts):
    x = hid_ref[0]
    L = x.shape[0]
    acc = None
    for k in range(_KS):
        a = jax.lax.dot_general(w1_ref[k], x, (((1,), (1,)), ((), ())),
                                preferred_element_type=jnp.float32)
        if k:
            a = jnp.roll(a, -k, axis=1)
        acc = a if acc is None else acc + a
    r = jnp.maximum(acc + b1_ref[...], 0.0)
    strided = jnp.dot(w2_ref[...], r,
                      preferred_element_type=jnp.float32) + b2_ref[0, 0]
    full = jnp.roll(strided, _KS - 1, axis=1)
    lane = jax.lax.broadcasted_iota(jnp.int32, (1, L), 1)
    full = jnp.where(lane < _KS - 1, _NEG, full)
    h = (full > 0.0).astype(jnp.float32)
    nb_ref[0] = jnp.sum(h, keepdims=True)
    hprev = jnp.where(lane < 1, 0.0, jnp.roll(h, 1, axis=1))
    adj_ref[0] = jnp.sum(h * hprev, keepdims=True)
    C = 128
    nchunk = L // C
    hb = h.astype(jnp.bfloat16)
    t128 = (jax.lax.broadcasted_iota(jnp.int32, (C, C), 0)
            < jax.lax.broadcasted_iota(jnp.int32, (C, C), 1)).astype(
                jnp.bfloat16)
    fine = jnp.concatenate(
        [jnp.dot(hb[:, j * C:(j + 1) * C], t128,
                 preferred_element_type=jnp.float32)
         for j in range(nchunk)], axis=1)
    bmask = (jax.lax.broadcasted_iota(jnp.int32, (L, nchunk), 0) // C
             == jax.lax.broadcasted_iota(jnp.int32, (L, nchunk), 1)).astype(
                 jnp.bfloat16)
    tot = jnp.dot(hb, bmask, preferred_element_type=jnp.float32)
    qmask = (jax.lax.broadcasted_iota(jnp.int32, (nchunk, L), 0)
             < jax.lax.broadcasted_iota(jnp.int32, (nchunk, L), 1) // C
             ).astype(jnp.bfloat16)
    offs = jnp.dot(tot.astype(jnp.bfloat16), qmask,
                   preferred_element_type=jnp.float32)
    seg = fine + offs
    x_hi = x.astype(jnp.bfloat16)
    x_lo = (x - x_hi.astype(jnp.float32)).astype(jnp.bfloat16)
    nbs = jnp.sum(h)
    for st in range(L // ts):
        sl = slice(st * ts, (st + 1) * ts)

        def _run(sl=sl, st=st):
            sidx = (jax.lax.broadcasted_iota(jnp.int32, (ts, 1), 0)
                    + st * ts).astype(jnp.float32)
            pmf = (sidx == seg).astype(jnp.float32)
            pm = pmf.astype(jnp.bfloat16)
            sums = (jnp.dot(pm, x_hi, preferred_element_type=jnp.float32)
                    + jnp.dot(pm, x_lo, preferred_element_type=jnp.float32))
            cnt = jnp.sum(pmf, axis=1, keepdims=True)
            pooled_ref[0, sl, :] = sums / jnp.maximum(cnt, 1.0) + pe_ref[sl, :]
            cnt_ref[0, sl] = cnt

        if st == 0:
            _run()
        else:
            pl.when(nbs >= float(st * ts))(_run)

            @pl.when(nbs < float(st * ts))
            def _skip(sl=sl):
                pooled_ref[0, sl, :] = pe_ref[sl, :]
                cnt_ref[0, sl] = jnp.zeros((ts, 1), jnp.float32)


def _stats_kernel(cnt_ref, nb_ref, adj_ref, out_ref):
    c = cnt_ref[...]
    valid = (c > 0.0).astype(jnp.float32)
    n_valid = jnp.maximum(jnp.sum(valid), 1.0)
    mean = jnp.sum(c * valid) / n_valid
    var = jnp.sum(valid * (c - mean) ** 2) / n_valid
    cv = jnp.sqrt(var) / jnp.maximum(mean, 1e-6)
    nb = jnp.sum(nb_ref[...])
    adj = jnp.sum(adj_ref[...])
    adj_pct = 100.0 * adj / jnp.maximum(nb, 1.0)
    lane = jax.lax.broadcasted_iota(jnp.int32, (1, 8), 1)
    out_ref[...] = jnp.where(
        lane == 0, nb, jnp.where(lane == 1, cv,
                                 jnp.where(lane == 2, adj_pct, 0.0)))


def kernel(hidden, W1, b1, W2, b2):
    B, L, D = hidden.shape
    H = W1.shape[0]
    HP = 128
    W1p = jnp.zeros((_KS, HP, D), jnp.float32).at[:, :H, :].set(
        jnp.transpose(W1, (2, 0, 1)))
    b1p = jnp.zeros((HP, 1), jnp.float32).at[:H, 0].set(b1)
    w2p = jnp.zeros((1, HP), jnp.float32).at[0, :H].set(W2[0, :, 0])
    b2r = b2.reshape(1, 1).astype(jnp.float32)
    pe = _pos_emb(L, D)

    TS = 128
    pooled, cnt3, nb, adj = pl.pallas_call(
        functools.partial(_fused_kernel, ts=TS),
        grid=(B,),
        in_specs=[
            pl.BlockSpec((1, L, D), lambda b: (b, 0, 0)),
            pl.BlockSpec((_KS, HP, D), lambda b: (0, 0, 0)),
            pl.BlockSpec((HP, 1), lambda b: (0, 0)),
            pl.BlockSpec((1, HP), lambda b: (0, 0)),
            pl.BlockSpec((1, 1), lambda b: (0, 0)),
            pl.BlockSpec((L, D), lambda b: (0, 0)),
        ],
        out_specs=[
            pl.BlockSpec((1, L, D), lambda b: (b, 0, 0)),
            pl.BlockSpec((1, L, 1), lambda b: (b, 0, 0)),
            pl.BlockSpec((1, 1, 1), lambda b: (b, 0, 0)),
            pl.BlockSpec((1, 1, 1), lambda b: (b, 0, 0)),
        ],
        out_shape=[
            jax.ShapeDtypeStruct((B, L, D), jnp.float32),
            jax.ShapeDtypeStruct((B, L, 1), jnp.float32),
            jax.ShapeDtypeStruct((B, 1, 1), jnp.float32),
            jax.ShapeDtypeStruct((B, 1, 1), jnp.float32),
        ],
    )(hidden, W1p, b1p, w2p, b2r, pe)

    counts = cnt3.reshape(B, L)
    stats = pl.pallas_call(
        _stats_kernel,
        out_shape=jax.ShapeDtypeStruct((1, 8), jnp.float32),
    )(counts, nb, adj)

    loss = jnp.asarray(0.0, dtype=jnp.float32)
    total_positions = jnp.asarray(float(B * L), dtype=jnp.float32)
    return (pooled, loss, stats[0, 0], total_positions, stats[0, 1],
            stats[0, 2])


# speedup vs baseline: 4.0329x; 1.1638x over previous
import functools

import numpy as np
import jax
import jax.numpy as jnp
from jax.experimental import pallas as pl

_KS = 3
_NEG = -10.0


def _pos_emb(L, D):
    pos = np.arange(L)[:, None].astype(np.float64)
    i = np.arange(D)[None, :].astype(np.float64)
    angle = pos / np.power(10000.0, (2.0 * (i // 2)) / D)
    pe = np.zeros((L, D), dtype=np.float64)
    pe[:, 0::2] = np.sin(angle[:, 0::2])
    pe[:, 1::2] = np.cos(angle[:, 1::2])
    return jnp.asarray(pe, dtype=jnp.float32)


def _fused_kernel(hid_ref, w1_ref, b1_ref, w2_ref, b2_ref, pe_ref,
                  pooled_ref, cnt_ref, nb_ref, adj_ref, *, ts):
    x = hid_ref[0]
    L = x.shape[0]
    acc = None
    for k in range(_KS):
        a = jax.lax.dot_general(w1_ref[k], x, (((1,), (1,)), ((), ())),
                                preferred_element_type=jnp.float32)
        if k:
            a = jnp.roll(a, -k, axis=1)
        acc = a if acc is None else acc + a
    r = jnp.maximum(acc + b1_ref[...], 0.0)
    strided = jnp.dot(w2_ref[...], r,
                      preferred_element_type=jnp.float32) + b2_ref[0, 0]
    full = jnp.roll(strided, _KS - 1, axis=1)
    lane = jax.lax.broadcasted_iota(jnp.int32, (1, L), 1)
    full = jnp.where(lane < _KS - 1, _NEG, full)
    h = (full > 0.0).astype(jnp.float32)
    nb_ref[0] = jnp.sum(h, keepdims=True)
    hprev = jnp.where(lane < 1, 0.0, jnp.roll(h, 1, axis=1))
    adj_ref[0] = jnp.sum(h * hprev, keepdims=True)
    C = 128
    nchunk = L // C
    hb = h.astype(jnp.bfloat16)
    t128 = (jax.lax.broadcasted_iota(jnp.int32, (C, C), 0)
            < jax.lax.broadcasted_iota(jnp.int32, (C, C), 1)).astype(
                jnp.bfloat16)
    fine = jnp.concatenate(
        [jnp.dot(hb[:, j * C:(j + 1) * C], t128,
                 preferred_element_type=jnp.float32)
         for j in range(nchunk)], axis=1)
    bmask = (jax.lax.broadcasted_iota(jnp.int32, (L, nchunk), 0) // C
             == jax.lax.broadcasted_iota(jnp.int32, (L, nchunk), 1)).astype(
                 jnp.bfloat16)
    tot = jnp.dot(hb, bmask, preferred_element_type=jnp.float32)
    qmask = (jax.lax.broadcasted_iota(jnp.int32, (nchunk, L), 0)
             < jax.lax.broadcasted_iota(jnp.int32, (nchunk, L), 1) // C
             ).astype(jnp.bfloat16)
    offs = jnp.dot(tot.astype(jnp.bfloat16), qmask,
                   preferred_element_type=jnp.float32)
    seg = fine + offs
    x_hi = x.astype(jnp.bfloat16)
    x_lo = (x - x_hi.astype(jnp.float32)).astype(jnp.bfloat16)
    nbs = jnp.sum(h)
    for st in range(L // ts):
        sl = slice(st * ts, (st + 1) * ts)

        def _run(sl=sl, st=st):
            sidx = (jax.lax.broadcasted_iota(jnp.int32, (ts, 1), 0)
                    + st * ts).astype(jnp.float32)
            pmf = (sidx == seg).astype(jnp.float32)
            pm = pmf.astype(jnp.bfloat16)
            sums = (jnp.dot(pm, x_hi, preferred_element_type=jnp.float32)
                    + jnp.dot(pm, x_lo, preferred_element_type=jnp.float32))
            cnt = jnp.sum(pmf, axis=1, keepdims=True)
            cnt_row = jax.lax.dot_general(
                jnp.ones((1, L), jnp.bfloat16), pm, (((1,), (1,)), ((), ())),
                preferred_element_type=jnp.float32)
            pooled_ref[0, sl, :] = sums / jnp.maximum(cnt, 1.0) + pe_ref[sl, :]
            cnt_ref[0, 0:1, sl] = cnt_row

        if st == 0:
            _run()
        else:
            pl.when(nbs >= float(st * ts))(_run)

            @pl.when(nbs < float(st * ts))
            def _skip(sl=sl):
                pooled_ref[0, sl, :] = pe_ref[sl, :]
                cnt_ref[0, 0:1, sl] = jnp.zeros((1, ts), jnp.float32)


def _stats_kernel(cnt_ref, nb_ref, adj_ref, out_ref):
    c = cnt_ref[...]
    valid = (c > 0.0).astype(jnp.float32)
    n_valid = jnp.maximum(jnp.sum(valid), 1.0)
    mean = jnp.sum(c * valid) / n_valid
    var = jnp.sum(valid * (c - mean) ** 2) / n_valid
    cv = jnp.sqrt(var) / jnp.maximum(mean, 1e-6)
    nb = jnp.sum(nb_ref[...])
    adj = jnp.sum(adj_ref[...])
    adj_pct = 100.0 * adj / jnp.maximum(nb, 1.0)
    lane = jax.lax.broadcasted_iota(jnp.int32, (1, 8), 1)
    out_ref[...] = jnp.where(
        lane == 0, nb, jnp.where(lane == 1, cv,
                                 jnp.where(lane == 2, adj_pct, 0.0)))


def kernel(hidden, W1, b1, W2, b2):
    B, L, D = hidden.shape
    H = W1.shape[0]
    HP = 128
    W1p = jnp.zeros((_KS, HP, D), jnp.float32).at[:, :H, :].set(
        jnp.transpose(W1, (2, 0, 1)))
    b1p = jnp.zeros((HP, 1), jnp.float32).at[:H, 0].set(b1)
    w2p = jnp.zeros((1, HP), jnp.float32).at[0, :H].set(W2[0, :, 0])
    b2r = b2.reshape(1, 1).astype(jnp.float32)
    pe = _pos_emb(L, D)

    TS = 128
    pooled, cnt3, nb, adj = pl.pallas_call(
        functools.partial(_fused_kernel, ts=TS),
        grid=(B,),
        in_specs=[
            pl.BlockSpec((1, L, D), lambda b: (b, 0, 0)),
            pl.BlockSpec((_KS, HP, D), lambda b: (0, 0, 0)),
            pl.BlockSpec((HP, 1), lambda b: (0, 0)),
            pl.BlockSpec((1, HP), lambda b: (0, 0)),
            pl.BlockSpec((1, 1), lambda b: (0, 0)),
            pl.BlockSpec((L, D), lambda b: (0, 0)),
        ],
        out_specs=[
            pl.BlockSpec((1, L, D), lambda b: (b, 0, 0)),
            pl.BlockSpec((1, 1, L), lambda b: (b, 0, 0)),
            pl.BlockSpec((1, 1, 1), lambda b: (b, 0, 0)),
            pl.BlockSpec((1, 1, 1), lambda b: (b, 0, 0)),
        ],
        out_shape=[
            jax.ShapeDtypeStruct((B, L, D), jnp.float32),
            jax.ShapeDtypeStruct((B, 1, L), jnp.float32),
            jax.ShapeDtypeStruct((B, 1, 1), jnp.float32),
            jax.ShapeDtypeStruct((B, 1, 1), jnp.float32),
        ],
    )(hidden, W1p, b1p, w2p, b2r, pe)

    counts = cnt3.reshape(B, L)
    stats = pl.pallas_call(
        _stats_kernel,
        out_shape=jax.ShapeDtypeStruct((1, 8), jnp.float32),
    )(counts, nb, adj)

    loss = jnp.asarray(0.0, dtype=jnp.float32)
    total_positions = jnp.asarray(float(B * L), dtype=jnp.float32)
    return (pooled, loss, stats[0, 0], total_positions, stats[0, 1],
            stats[0, 2])


# speedup vs baseline: 4.2829x; 1.0620x over previous
import functools

import numpy as np
import jax
import jax.numpy as jnp
from jax.experimental import pallas as pl
from jax.experimental.pallas import tpu as pltpu

_KS = 3
_NEG = -10.0


def _pos_emb(L, D):
    pos = np.arange(L)[:, None].astype(np.float64)
    i = np.arange(D)[None, :].astype(np.float64)
    angle = pos / np.power(10000.0, (2.0 * (i // 2)) / D)
    pe = np.zeros((L, D), dtype=np.float64)
    pe[:, 0::2] = np.sin(angle[:, 0::2])
    pe[:, 1::2] = np.cos(angle[:, 1::2])
    return jnp.asarray(pe, dtype=jnp.float32)


def _fused_kernel(hid_ref, w1_ref, b1_ref, w2_ref, b2_ref, pe_ref,
                  pooled_ref, stats_ref, cnt_ref, acc_ref, *, ts):
    bb = pl.program_id(0)
    lane8 = jax.lax.broadcasted_iota(jnp.int32, (1, 8), 1)

    @pl.when(bb == 0)
    def _init():
        acc_ref[...] = jnp.zeros((1, 8), jnp.float32)
    x = hid_ref[0]
    L = x.shape[0]
    acc = None
    for k in range(_KS):
        a = jax.lax.dot_general(w1_ref[k], x, (((1,), (1,)), ((), ())),
                                preferred_element_type=jnp.float32)
        if k:
            a = jnp.roll(a, -k, axis=1)
        acc = a if acc is None else acc + a
    r = jnp.maximum(acc + b1_ref[...], 0.0)
    strided = jnp.dot(w2_ref[...], r,
                      preferred_element_type=jnp.float32) + b2_ref[0, 0]
    full = jnp.roll(strided, _KS - 1, axis=1)
    lane = jax.lax.broadcasted_iota(jnp.int32, (1, L), 1)
    full = jnp.where(lane < _KS - 1, _NEG, full)
    h = (full > 0.0).astype(jnp.float32)
    nbs = jnp.sum(h)
    hprev = jnp.where(lane < 1, 0.0, jnp.roll(h, 1, axis=1))
    adjs = jnp.sum(h * hprev)
    acc_ref[...] = acc_ref[...] + jnp.where(
        lane8 == 0, nbs, jnp.where(lane8 == 1, adjs, 0.0))
    C = 128
    nchunk = L // C
    hb = h.astype(jnp.bfloat16)
    t128 = (jax.lax.broadcasted_iota(jnp.int32, (C, C), 0)
            < jax.lax.broadcasted_iota(jnp.int32, (C, C), 1)).astype(
                jnp.bfloat16)
    fine = jnp.concatenate(
        [jnp.dot(hb[:, j * C:(j + 1) * C], t128,
                 preferred_element_type=jnp.float32)
         for j in range(nchunk)], axis=1)
    bmask = (jax.lax.broadcasted_iota(jnp.int32, (L, nchunk), 0) // C
             == jax.lax.broadcasted_iota(jnp.int32, (L, nchunk), 1)).astype(
                 jnp.bfloat16)
    tot = jnp.dot(hb, bmask, preferred_element_type=jnp.float32)
    qmask = (jax.lax.broadcasted_iota(jnp.int32, (nchunk, L), 0)
             < jax.lax.broadcasted_iota(jnp.int32, (nchunk, L), 1) // C
             ).astype(jnp.bfloat16)
    offs = jnp.dot(tot.astype(jnp.bfloat16), qmask,
                   preferred_element_type=jnp.float32)
    seg = fine + offs
    x_hi = x.astype(jnp.bfloat16)
    x_lo = (x - x_hi.astype(jnp.float32)).astype(jnp.bfloat16)
    for st in range(L // ts):
        sl = slice(st * ts, (st + 1) * ts)

        def _run(sl=sl, st=st):
            sidx = (jax.lax.broadcasted_iota(jnp.int32, (ts, 1), 0)
                    + st * ts).astype(jnp.float32)
            pmf = (sidx == seg).astype(jnp.float32)
            pm = pmf.astype(jnp.bfloat16)
            sums = (jnp.dot(pm, x_hi, preferred_element_type=jnp.float32)
                    + jnp.dot(pm, x_lo, preferred_element_type=jnp.float32))
            cnt = jnp.sum(pmf, axis=1, keepdims=True)
            cnt_row = jax.lax.dot_general(
                jnp.ones((1, L), jnp.bfloat16), pm, (((1,), (1,)), ((), ())),
                preferred_element_type=jnp.float32)
            pooled_ref[0, sl, :] = sums / jnp.maximum(cnt, 1.0) + pe_ref[sl, :]
            cnt_ref[pl.ds(bb * 8, 1), sl] = cnt_row

        if st == 0:
            _run()
        else:
            pl.when(nbs >= float(st * ts))(_run)

            @pl.when(nbs < float(st * ts))
            def _skip(sl=sl):
                pooled_ref[0, sl, :] = pe_ref[sl, :]
                cnt_ref[pl.ds(bb * 8, 1), sl] = jnp.zeros((1, ts), jnp.float32)

    @pl.when(bb == pl.num_programs(0) - 1)
    def _stats():
        craw = cnt_ref[...]
        rsel = (jax.lax.broadcasted_iota(jnp.int32, craw.shape, 0) % 8 == 0)
        c = jnp.where(rsel, craw, 0.0)
        valid = (c > 0.0).astype(jnp.float32)
        n_valid = jnp.maximum(jnp.sum(valid), 1.0)
        mean = jnp.sum(c * valid) / n_valid
        var = jnp.sum(valid * (c - mean) ** 2) / n_valid
        cv = jnp.sqrt(var) / jnp.maximum(mean, 1e-6)
        vals = acc_ref[...]
        nb = jnp.sum(jnp.where(lane8 == 0, vals, 0.0))
        adj = jnp.sum(jnp.where(lane8 == 1, vals, 0.0))
        adj_pct = 100.0 * adj / jnp.maximum(nb, 1.0)
        stats_ref[...] = jnp.where(
            lane8 == 0, nb, jnp.where(lane8 == 1, cv,
                                      jnp.where(lane8 == 2, adj_pct, 0.0)))


def kernel(hidden, W1, b1, W2, b2):
    B, L, D = hidden.shape
    H = W1.shape[0]
    HP = 128
    W1p = jnp.zeros((_KS, HP, D), jnp.float32).at[:, :H, :].set(
        jnp.transpose(W1, (2, 0, 1)))
    b1p = jnp.zeros((HP, 1), jnp.float32).at[:H, 0].set(b1)
    w2p = jnp.zeros((1, HP), jnp.float32).at[0, :H].set(W2[0, :, 0])
    b2r = b2.reshape(1, 1).astype(jnp.float32)
    pe = _pos_emb(L, D)

    TS = 128
    pooled, stats = pl.pallas_call(
        functools.partial(_fused_kernel, ts=TS),
        grid=(B,),
        in_specs=[
            pl.BlockSpec((1, L, D), lambda b: (b, 0, 0)),
            pl.BlockSpec((_KS, HP, D), lambda b: (0, 0, 0)),
            pl.BlockSpec((HP, 1), lambda b: (0, 0)),
            pl.BlockSpec((1, HP), lambda b: (0, 0)),
            pl.BlockSpec((1, 1), lambda b: (0, 0)),
            pl.BlockSpec((L, D), lambda b: (0, 0)),
        ],
        out_specs=[
            pl.BlockSpec((1, L, D), lambda b: (b, 0, 0)),
            pl.BlockSpec((1, 8), lambda b: (0, 0)),
        ],
        out_shape=[
            jax.ShapeDtypeStruct((B, L, D), jnp.float32),
            jax.ShapeDtypeStruct((1, 8), jnp.float32),
        ],
        scratch_shapes=[
            pltpu.VMEM((8 * B, L), jnp.float32),
            pltpu.VMEM((1, 8), jnp.float32),
        ],
    )(hidden, W1p, b1p, w2p, b2r, pe)

    loss = jnp.asarray(0.0, dtype=jnp.float32)
    total_positions = jnp.asarray(float(B * L), dtype=jnp.float32)
    return (pooled, loss, stats[0, 0], total_positions, stats[0, 1],
            stats[0, 2])
